# baseline (speedup 1.0000x reference)
import functools

import jax
import jax.numpy as jnp
import numpy as np
from jax.experimental import pallas as pl
from jax.experimental.pallas import tpu as pltpu

_L = 5
_AVG_D = float(np.log(33.0))


def _readout_body(u1_ref, u2_ref, g_ref, bt_ref, w2_ref, b2_ref, o_ref,
                  acc_ref, *, nb, total):
    phase = pl.program_id(0)
    b = pl.program_id(1)
    u1v = u1_ref[...]
    u2v = u2_ref[...]

    @pl.when((phase == 0) & (b == 0))
    def _init():
        acc_ref[...] = jnp.zeros_like(acc_ref)

    @pl.when(phase == 0)
    def _p0():
        acc_ref[0:1, :] += jnp.sum(u1v, axis=0, keepdims=True)
        acc_ref[1:2, :] += jnp.sum(u2v, axis=0, keepdims=True)

        @pl.when(b == nb - 1)
        def _finish_mean():
            acc_ref[4:5, :] = acc_ref[0:1, :] / total
            acc_ref[5:6, :] = acc_ref[1:2, :] / total

    @pl.when(phase == 1)
    def _p1():
        mu1 = acc_ref[4:5, :]
        mu2 = acc_ref[5:6, :]
        acc_ref[2:3, :] += jnp.sum((u1v - mu1) ** 2, axis=0, keepdims=True)
        acc_ref[3:4, :] += jnp.sum((u2v - mu2) ** 2, axis=0, keepdims=True)

        @pl.when(b == nb - 1)
        def _finish_var():
            acc_ref[6:7, :] = jax.lax.rsqrt(acc_ref[2:3, :] / total + 1e-5)
            acc_ref[7:8, :] = jax.lax.rsqrt(acc_ref[3:4, :] / total + 1e-5)

    @pl.when(phase == 2)
    def _p2():
        g = g_ref[...]
        bt = bt_ref[...]
        w2 = w2_ref[...]

        def head(u, mu, rstd):
            hn = (u - mu) * rstd * g + bt
            hn = jnp.maximum(hn, 0.0)
            return jnp.sum(hn * w2, axis=1)

        d = (head(u1v, acc_ref[4:5, :], acc_ref[6:7, :])
             + head(u2v, acc_ref[5:6, :], acc_ref[7:8, :])
             + 2.0 * b2_ref[0, 0])
        o_ref[...] = jnp.logaddexp(d, 0.0)[:, None]


def _readout(u1, u2, g1, bt1, W2, b2):
    total, pd = u1.shape
    nb = 10
    bs = total // nb
    assert bs * nb == total and bs % 8 == 0
    body = functools.partial(_readout_body, nb=nb, total=float(total))
    out = pl.pallas_call(
        body,
        grid=(3, nb),
        in_specs=[
            pl.BlockSpec((bs, pd), lambda p, b: (b, 0)),
            pl.BlockSpec((bs, pd), lambda p, b: (b, 0)),
            pl.BlockSpec((1, pd), lambda p, b: (0, 0)),
            pl.BlockSpec((1, pd), lambda p, b: (0, 0)),
            pl.BlockSpec((1, pd), lambda p, b: (0, 0)),
            pl.BlockSpec((1, 1), lambda p, b: (0, 0)),
        ],
        out_specs=pl.BlockSpec((bs, 1), lambda p, b: (b, 0)),
        out_shape=jax.ShapeDtypeStruct((total, 1), jnp.float32),
        scratch_shapes=[pltpu.VMEM((8, pd), jnp.float32)],
        compiler_params=pltpu.CompilerParams(
            dimension_semantics=("arbitrary", "arbitrary")),
    )(u1, u2, g1.reshape(1, pd), bt1.reshape(1, pd),
      W2.reshape(1, pd), b2.reshape(1, 1))
    return out


def kernel(x, edge_index, pairwise_indices, mask, params):
    del mask
    src, dst = edge_index[0], edge_index[1]
    n = x.shape[0]
    h = x
    deg = jax.ops.segment_sum(jnp.ones((src.shape[0],), jnp.float32), dst,
                              num_segments=n)
    degc = jnp.maximum(deg, 1.0)
    logd = jnp.log(deg + 1.0)
    s_amp = (logd / _AVG_D)[:, None]
    s_att = (_AVG_D / jnp.where(deg > 0, logd, 1.0))[:, None]
    has = (deg > 0)[:, None]
    for l in range(_L):
        Wp, bp = params["W_pre"][l], params["b_pre"][l]
        Wq, bq = params["W_post"][l], params["b_post"][l]
        a = h @ Wp[:128]
        b = h @ Wp[128:] + bp
        m = jax.nn.relu(a[src] + b[dst])
        ssum = jax.ops.segment_sum(m, dst, num_segments=n)
        mean = ssum / degc[:, None]
        sq = jax.ops.segment_sum(m * m, dst, num_segments=n) / degc[:, None]
        std = jnp.sqrt(jax.nn.relu(sq - mean * mean) + 1e-5)
        mx = jnp.where(has, jax.ops.segment_max(m, dst, num_segments=n), 0.0)
        mn = jnp.where(has, jax.ops.segment_min(m, dst, num_segments=n), 0.0)
        agg = jnp.concatenate([mean, mx, mn, std], axis=1)
        scaled = jnp.concatenate([agg, agg * s_amp, agg * s_att], axis=1)
        h = h + jnp.concatenate([h, scaled], axis=1) @ Wq + bq

    W1, b1, g1, bt1, W2, b2 = params["dn"]
    p = h @ W1[:128]
    q = h @ W1[128:] + b1
    i, j = pairwise_indices[0], pairwise_indices[1]
    u1 = p[i] + q[j]
    u2 = p[j] + q[i]
    return _readout(u1, u2, g1, bt1, W2, b2)
